# baseline (device time: 17324 ns/iter reference)
import jax
import jax.numpy as jnp
from jax import lax
from jax.experimental import pallas as pl
from jax.experimental.pallas import tpu as pltpu

N_DEV = 16


def kernel(x, t_emb, W_scale, W_shift):
    b, s, c_local = x.shape
    c_total = float(N_DEV * c_local)

    def body(x_ref, t_ref, ws_ref, wsh_ref, out_ref,
             gather_ref, send_sems, recv_sems, exit_sem):
        my_pos = lax.axis_index("i")

        xs = x_ref[...]
        s1 = jnp.sum(xs, axis=-1)
        s2 = jnp.sum(xs * xs, axis=-1)
        gather_ref[pl.ds(my_pos, 1)] = jnp.stack([s1, s2])[None]

        barrier_sem = pltpu.get_barrier_semaphore()
        for d in range(1, N_DEV):
            peer = (my_pos + d) % N_DEV
            pl.semaphore_signal(barrier_sem, inc=1, device_id=(peer,),
                                device_id_type=pl.DeviceIdType.MESH)
        pl.semaphore_wait(barrier_sem, N_DEV - 1)

        rdmas = []
        for d in range(1, N_DEV):
            dst = (my_pos + d) % N_DEV
            rdma = pltpu.make_async_remote_copy(
                src_ref=gather_ref.at[my_pos],
                dst_ref=gather_ref.at[my_pos],
                send_sem=send_sems.at[d - 1],
                recv_sem=recv_sems.at[d - 1],
                device_id=(dst,),
                device_id_type=pl.DeviceIdType.MESH,
            )
            rdma.start()
            rdmas.append(rdma)
        for rdma in rdmas:
            rdma.wait()

        totals = jnp.sum(gather_ref[...], axis=0)

        for d in range(1, N_DEV):
            peer = (my_pos + d) % N_DEV
            pl.semaphore_signal(exit_sem, inc=1, device_id=(peer,),
                                device_id_type=pl.DeviceIdType.MESH)

        mean = totals[0] / c_total
        var = totals[1] / c_total - mean * mean
        inv = lax.rsqrt(var + 1e-5)
        scale = jnp.dot(t_ref[...], ws_ref[...],
                        preferred_element_type=jnp.float32)
        shift = jnp.dot(t_ref[...], wsh_ref[...],
                        preferred_element_type=jnp.float32)
        h = (xs - mean[:, :, None]) * inv[:, :, None]
        out_ref[...] = h * (1.0 + scale[:, None, :]) + shift[:, None, :]

        pl.semaphore_wait(exit_sem, N_DEV - 1)

    return pl.pallas_call(
        body,
        out_shape=jax.ShapeDtypeStruct((b, s, c_local), jnp.float32),
        in_specs=[pl.BlockSpec(memory_space=pltpu.VMEM)] * 4,
        out_specs=pl.BlockSpec(memory_space=pltpu.VMEM),
        scratch_shapes=[
            pltpu.VMEM((N_DEV, 2, b, s), jnp.float32),
            pltpu.SemaphoreType.DMA((N_DEV - 1,)),
            pltpu.SemaphoreType.DMA((N_DEV - 1,)),
            pltpu.SemaphoreType.REGULAR,
        ],
        compiler_params=pltpu.CompilerParams(collective_id=0),
    )(x, t_emb, W_scale, W_shift)
